# baseline (speedup 1.0000x reference)
import functools

import jax
import jax.numpy as jnp
from jax import lax
from jax.experimental import pallas as pl
from jax.experimental.pallas import tpu as pltpu
from jax.experimental.pallas import tpu_sc as plsc

_VOCAB = 1000000
_EMB = 64
_RANK = 1
_WORLD = 4
_NUM_PER_RANK = _VOCAB // _WORLD
_LOWER = _RANK * _NUM_PER_RANK
_UPPER = (_RANK + 1) * _NUM_PER_RANK

_BATCH = 4096
_SEQ = 200
_TOTAL = _BATCH * _SEQ

_NC = 2
_NS = 16
_NW = _NC * _NS
_PER_W = _TOTAL // _NW
_CHUNK = 800
_NCHUNK = _PER_W // _CHUNK
_GROUPS = _CHUNK // 16
_NZERO = _NW * _CHUNK


def _body(
    ids_hbm, table_hbm, out_hbm,
    raw_a, raw_b, idx_a, idx_b, rows_a, rows_b,
    sem_ga, sem_gb, sem_oa, sem_ob,
):
    wid = lax.axis_index("s") * _NC + lax.axis_index("c")
    lane = lax.iota(jnp.int32, 16)
    zbase = _NUM_PER_RANK + wid * _CHUNK

    def stage(g, raw_v, idx_v):
        base = wid * _PER_W + g * _CHUNK
        pltpu.sync_copy(ids_hbm.at[pl.ds(base, _CHUNK)], raw_v)

        def xform(i, _):
            v = raw_v[pl.ds(i * 16, 16)]
            valid = (v >= _LOWER) & (v < _UPPER)
            zrow = zbase + i * 16 + lane
            idx_v[pl.ds(i * 16, 16)] = jnp.where(valid, v - _LOWER, zrow)
            return _

        lax.fori_loop(0, _GROUPS, xform, None)

    def fire_gather(idx_v, rows_v, sem):
        pltpu.async_copy(table_hbm.at[idx_v], rows_v, sem)

    def wait_gather(idx_v, rows_v, sem):
        pltpu.make_async_copy(table_hbm.at[idx_v], rows_v, sem).wait()

    def fire_out(g, rows_v, sem):
        base = wid * _PER_W + g * _CHUNK
        pltpu.async_copy(rows_v, out_hbm.at[pl.ds(base, _CHUNK)], sem)

    def wait_out(g, rows_v, sem):
        base = wid * _PER_W + g * _CHUNK
        pltpu.make_async_copy(rows_v, out_hbm.at[pl.ds(base, _CHUNK)], sem).wait()

    stage(0, raw_a, idx_a)
    fire_gather(idx_a, rows_a, sem_ga)
    stage(1, raw_b, idx_b)
    fire_gather(idx_b, rows_b, sem_gb)
    wait_gather(idx_a, rows_a, sem_ga)
    fire_out(0, rows_a, sem_oa)

    def pipe(i, _):
        ga = 2 * i
        gb = 2 * i + 1
        stage(ga, raw_a, idx_a)
        wait_out(ga - 2, rows_a, sem_oa)
        fire_gather(idx_a, rows_a, sem_ga)
        wait_gather(idx_b, rows_b, sem_gb)
        fire_out(gb - 2, rows_b, sem_ob)
        stage(gb, raw_b, idx_b)
        wait_out(gb - 2, rows_b, sem_ob)
        fire_gather(idx_b, rows_b, sem_gb)
        wait_gather(idx_a, rows_a, sem_ga)
        fire_out(ga, rows_a, sem_oa)
        return _

    lax.fori_loop(1, _NCHUNK // 2, pipe, None)

    wait_gather(idx_b, rows_b, sem_gb)
    fire_out(_NCHUNK - 1, rows_b, sem_ob)
    wait_out(_NCHUNK - 2, rows_a, sem_oa)
    wait_out(_NCHUNK - 1, rows_b, sem_ob)


@jax.jit
def kernel(input_ids, embedding_table):
    ids_flat = input_ids.reshape(_TOTAL)
    table_ext = jnp.concatenate(
        [embedding_table, jnp.zeros((_NZERO, _EMB), jnp.float32)], axis=0
    )
    out = pl.kernel(
        _body,
        out_type=jax.ShapeDtypeStruct((_TOTAL, _EMB), jnp.float32),
        mesh=plsc.VectorSubcoreMesh(core_axis_name="c", subcore_axis_name="s"),
        scratch_types=[
            pltpu.VMEM((_CHUNK,), jnp.int32),
            pltpu.VMEM((_CHUNK,), jnp.int32),
            pltpu.VMEM((_CHUNK,), jnp.int32),
            pltpu.VMEM((_CHUNK,), jnp.int32),
            pltpu.VMEM((_CHUNK, _EMB), jnp.float32),
            pltpu.VMEM((_CHUNK, _EMB), jnp.float32),
            pltpu.SemaphoreType.DMA,
            pltpu.SemaphoreType.DMA,
            pltpu.SemaphoreType.DMA,
            pltpu.SemaphoreType.DMA,
        ],
        compiler_params=pltpu.CompilerParams(
            needs_layout_passes=False,
            use_tc_tiling_on_sc=False,
            disable_bounds_checks=True,
        ),
    )(ids_flat, table_ext)
    return out.reshape(_BATCH, _SEQ, _EMB)


# speedup vs baseline: 1.0012x; 1.0012x over previous
import jax
import jax.numpy as jnp
from jax import lax
from jax.experimental import pallas as pl
from jax.experimental.pallas import tpu as pltpu
from jax.experimental.pallas import tpu_sc as plsc

_VOCAB = 1000000
_EMB = 64
_RANK = 1
_WORLD = 4
_NUM_PER_RANK = _VOCAB // _WORLD
_LOWER = _RANK * _NUM_PER_RANK
_UPPER = (_RANK + 1) * _NUM_PER_RANK

_BATCH = 4096
_SEQ = 200
_TOTAL = _BATCH * _SEQ

_NC = 2
_NS = 16
_NW = _NC * _NS
_PER_W = _TOTAL // _NW
_CHUNK = 800
_NCHUNK = _PER_W // _CHUNK
_GROUPS = _CHUNK // 16
_NZERO = _NW * _CHUNK


def _body(
    ids_hbm, table_hbm, out_hbm,
    raw_a, raw_b, idx_a, idx_b, rows_a, rows_b,
    sem_ga, sem_gb, sem_oa, sem_ob,
):
    wid = lax.axis_index("s") * _NC + lax.axis_index("c")
    lane = lax.iota(jnp.int32, 16)
    zbase = _NUM_PER_RANK + wid * _CHUNK

    def stage(g, raw_v, idx_v):
        base = wid * _PER_W + g * _CHUNK
        pltpu.sync_copy(ids_hbm.at[pl.ds(base, _CHUNK)], raw_v)

        def xform(i, _):
            v = raw_v[pl.ds(i * 16, 16)]
            valid = (v >= _LOWER) & (v < _UPPER)
            zrow = zbase + i * 16 + lane
            idx_v[pl.ds(i * 16, 16)] = jnp.where(valid, v - _LOWER, zrow)
            return _

        lax.fori_loop(0, _GROUPS, xform, None)

    def fire_gather(idx_v, rows_v, sem):
        pltpu.async_copy(table_hbm.at[idx_v], rows_v, sem)

    def wait_gather(idx_v, rows_v, sem):
        pltpu.make_async_copy(table_hbm.at[idx_v], rows_v, sem).wait()

    def fire_out(g, rows_v, sem):
        base = wid * _PER_W + g * _CHUNK
        pltpu.async_copy(rows_v, out_hbm.at[pl.ds(base, _CHUNK)], sem)

    def wait_out(g, rows_v, sem):
        base = wid * _PER_W + g * _CHUNK
        pltpu.make_async_copy(rows_v, out_hbm.at[pl.ds(base, _CHUNK)], sem).wait()

    stage(0, raw_a, idx_a)
    fire_gather(idx_a, rows_a, sem_ga)
    stage(1, raw_b, idx_b)
    fire_gather(idx_b, rows_b, sem_gb)
    wait_gather(idx_a, rows_a, sem_ga)
    fire_out(0, rows_a, sem_oa)

    def pipe(i, _):
        ga = 2 * i
        gb = 2 * i + 1
        stage(ga, raw_a, idx_a)
        wait_out(ga - 2, rows_a, sem_oa)
        fire_gather(idx_a, rows_a, sem_ga)
        wait_gather(idx_b, rows_b, sem_gb)
        fire_out(gb - 2, rows_b, sem_ob)
        stage(gb, raw_b, idx_b)
        wait_out(gb - 2, rows_b, sem_ob)
        fire_gather(idx_b, rows_b, sem_gb)
        wait_gather(idx_a, rows_a, sem_ga)
        fire_out(ga, rows_a, sem_oa)
        return _

    lax.fori_loop(1, _NCHUNK // 2, pipe, None)

    wait_gather(idx_b, rows_b, sem_gb)
    fire_out(_NCHUNK - 1, rows_b, sem_ob)
    wait_out(_NCHUNK - 2, rows_a, sem_oa)
    wait_out(_NCHUNK - 1, rows_b, sem_ob)


@jax.jit
def kernel(input_ids, embedding_table):
    ids_flat = input_ids.reshape(_TOTAL)
    table_ext = jnp.concatenate(
        [embedding_table, jnp.zeros((_NZERO, _EMB), jnp.float32)], axis=0
    )
    out = pl.kernel(
        _body,
        out_type=jax.ShapeDtypeStruct((_TOTAL, _EMB), jnp.float32),
        mesh=plsc.VectorSubcoreMesh(core_axis_name="c", subcore_axis_name="s"),
        scratch_types=[
            pltpu.VMEM((_CHUNK,), jnp.int32),
            pltpu.VMEM((_CHUNK,), jnp.int32),
            pltpu.VMEM((_CHUNK,), jnp.int32),
            pltpu.VMEM((_CHUNK,), jnp.int32),
            pltpu.VMEM((_CHUNK, _EMB), jnp.float32),
            pltpu.VMEM((_CHUNK, _EMB), jnp.float32),
            pltpu.SemaphoreType.DMA,
            pltpu.SemaphoreType.DMA,
            pltpu.SemaphoreType.DMA,
            pltpu.SemaphoreType.DMA,
        ],
        compiler_params=pltpu.CompilerParams(
            needs_layout_passes=False,
            use_tc_tiling_on_sc=False,
            disable_bounds_checks=True,
        ),
    )(ids_flat, table_ext)
    return out.reshape(_BATCH, _SEQ, _EMB)
